# Initial kernel scaffold; baseline (speedup 1.0000x reference)
import functools

import jax
import jax.numpy as jnp
from jax import lax
from jax.experimental import pallas as pl
from jax.experimental.pallas import tpu as pltpu
from jax.experimental.pallas import tpu_sc as plsc

N = 10000
E = 320000
D = 128
NP = 10240
NC = 2
NS = 16
NW = NC * NS
CHUNK = 80
INNER = 5
EPW = E // NW
ROWS_PER_W = EPW // CHUNK
OUTER = ROWS_PER_W // INNER
RPS = NP // NS


def _make_sc_agg(with_deg):
    out_type = [jax.ShapeDtypeStruct((NC, NP, D), jnp.float32)]
    scratch = [
        pltpu.VMEM((INNER, CHUNK), jnp.int32),
        pltpu.VMEM((INNER, CHUNK), jnp.int32),
        pltpu.VMEM((INNER * CHUNK, D), jnp.float32),
        pltpu.VMEM_SHARED((NP, D), jnp.float32),
        pltpu.SemaphoreType.DMA,
    ]
    if with_deg:
        out_type.append(jax.ShapeDtypeStruct((NC, NP), jnp.float32))
        scratch += [
            pltpu.VMEM((CHUNK,), jnp.float32),
            pltpu.VMEM_SHARED((NP,), jnp.float32),
        ]

    mesh = plsc.VectorSubcoreMesh(core_axis_name="c", subcore_axis_name="s")

    def body(*refs):
        if with_deg:
            (table, src2, dst2, zrows, zdeg, out_agg, out_deg,
             src_v, dst_v, rows_v, agg_sh, sem, ones_v, deg_sh) = refs
        else:
            (table, src2, dst2, zrows, out_agg,
             src_v, dst_v, rows_v, agg_sh, sem) = refs
        c = lax.axis_index("c")
        s = lax.axis_index("s")
        w = s * NC + c

        pltpu.sync_copy(zrows, agg_sh.at[pl.ds(s * RPS, RPS)])
        if with_deg:
            pltpu.sync_copy(zdeg, deg_sh.at[pl.ds(s * RPS, RPS)])
            for i in range(CHUNK // 16):
                ones_v[pl.ds(i * 16, 16)] = jnp.ones((16,), jnp.float32)
        plsc.subcore_barrier()

        row0 = w * ROWS_PER_W

        def outer(g, carry):
            r = row0 + g * INNER
            pltpu.sync_copy(src2.at[pl.ds(r, INNER)], src_v)
            pltpu.sync_copy(dst2.at[pl.ds(r, INNER)], dst_v)
            cps = []
            for j in range(INNER):
                cps.append(pltpu.async_copy(
                    table.at[src_v.at[j]],
                    rows_v.at[pl.ds(j * CHUNK, CHUNK)], sem))
            for j in range(INNER):
                cps[j].wait()
                pltpu.sync_copy(rows_v.at[pl.ds(j * CHUNK, CHUNK)],
                                agg_sh.at[dst_v.at[j]], add=True)
                if with_deg:
                    pltpu.sync_copy(ones_v, deg_sh.at[dst_v.at[j]], add=True)
            return carry

        lax.fori_loop(0, OUTER, outer, 0)
        plsc.subcore_barrier()
        pltpu.sync_copy(agg_sh.at[pl.ds(s * RPS, RPS)],
                        out_agg.at[c, pl.ds(s * RPS, RPS)])
        if with_deg:
            pltpu.sync_copy(deg_sh.at[pl.ds(s * RPS, RPS)],
                            out_deg.at[c, pl.ds(s * RPS, RPS)])

    return pl.kernel(body, out_type=tuple(out_type) if with_deg else out_type[0],
                     mesh=mesh, scratch_types=scratch)


_SC_AGG_DEG = _make_sc_agg(True)
_SC_AGG = _make_sc_agg(False)


def _make_combine(relu):
    R = 1024
    G = NP // R

    def body(x_ref, a0_ref, a1_ref, d0_ref, d1_ref, ws_ref, wn_ref, b_ref, o_ref):
        deg = jnp.maximum(d0_ref[...] + d1_ref[...], 1.0)
        mean = (a0_ref[...] + a1_ref[...]) / deg
        y = (jnp.dot(x_ref[...], ws_ref[...], preferred_element_type=jnp.float32)
             + jnp.dot(mean, wn_ref[...], preferred_element_type=jnp.float32)
             + b_ref[...])
        if relu:
            y = jnp.maximum(y, 0.0)
        o_ref[...] = y

    return pl.pallas_call(
        body,
        grid=(G,),
        in_specs=[
            pl.BlockSpec((R, D), lambda i: (i, 0)),
            pl.BlockSpec((R, D), lambda i: (i, 0)),
            pl.BlockSpec((R, D), lambda i: (i, 0)),
            pl.BlockSpec((R, 1), lambda i: (i, 0)),
            pl.BlockSpec((R, 1), lambda i: (i, 0)),
            pl.BlockSpec((D, D), lambda i: (0, 0)),
            pl.BlockSpec((D, D), lambda i: (0, 0)),
            pl.BlockSpec((1, D), lambda i: (0, 0)),
        ],
        out_specs=pl.BlockSpec((R, D), lambda i: (i, 0)),
        out_shape=jax.ShapeDtypeStruct((NP, D), jnp.float32),
    )


_COMBINE_RELU = _make_combine(True)
_COMBINE = _make_combine(False)


def kernel(x, edge_index, W1_self, W1_neigh, b1, W2_self, W2_neigh, b2):
    x = x.astype(jnp.float32)
    ei = edge_index.astype(jnp.int32)
    src2 = ei[0].reshape(E // CHUNK, CHUNK)
    dst2 = ei[1].reshape(E // CHUNK, CHUNK)
    xp = jnp.pad(x, ((0, NP - N), (0, 0)))
    zrows = jnp.zeros((RPS, D), jnp.float32)
    zdeg = jnp.zeros((RPS,), jnp.float32)

    agg1, deg = _SC_AGG_DEG(xp, src2, dst2, zrows, zdeg)
    d0 = deg[0][:, None]
    d1 = deg[1][:, None]
    h = _COMBINE_RELU(xp, agg1[0], agg1[1], d0, d1,
                      W1_self, W1_neigh, b1.reshape(1, D))
    agg2 = _SC_AGG(h, src2, dst2, zrows)
    out = _COMBINE(h, agg2[0], agg2[1], d0, d1,
                   W2_self, W2_neigh, b2.reshape(1, D))
    return out[:N]


# speedup vs baseline: 4.9067x; 4.9067x over previous
import jax
import jax.numpy as jnp
from jax import lax
from jax.experimental import pallas as pl
from jax.experimental.pallas import tpu as pltpu
from jax.experimental.pallas import tpu_sc as plsc

N = 10000
E = 320000
D = 128
SPL = 64
NP = 10240
NC = 2
NS = 16
CHUNK = 80
INNER = 8
EP = 327680
ROWS_PER_S = EP // NS // CHUNK
OUTER = ROWS_PER_S // INNER
RPS = NP // NS


def _make_sc_agg(with_deg):
    out_type = [jax.ShapeDtypeStruct((NC, NP, SPL), jnp.float32)]
    scratch = [
        pltpu.VMEM((INNER, CHUNK), jnp.int32),
        pltpu.VMEM((INNER, CHUNK), jnp.int32),
        pltpu.VMEM((INNER * CHUNK, SPL), jnp.float32),
        pltpu.VMEM_SHARED((NP, SPL), jnp.float32),
        pltpu.SemaphoreType.DMA,
    ]
    if with_deg:
        out_type.append(jax.ShapeDtypeStruct((NP,), jnp.float32))
        scratch += [
            pltpu.VMEM((CHUNK,), jnp.float32),
            pltpu.VMEM_SHARED((NP,), jnp.float32),
        ]

    mesh = plsc.VectorSubcoreMesh(core_axis_name="c", subcore_axis_name="s")

    def body(*refs):
        if with_deg:
            (table3, src2, dst2, zrows, zdeg, out_agg, out_deg,
             src_v, dst_v, rows_v, agg_sh, sem, ones_v, deg_sh) = refs
        else:
            (table3, src2, dst2, zrows, out_agg,
             src_v, dst_v, rows_v, agg_sh, sem) = refs
        c = lax.axis_index("c")
        s = lax.axis_index("s")

        pltpu.sync_copy(zrows, agg_sh.at[pl.ds(s * RPS, RPS)])
        if with_deg:
            pltpu.sync_copy(zdeg, deg_sh.at[pl.ds(s * RPS, RPS)])
            for i in range(CHUNK // 16):
                ones_v[pl.ds(i * 16, 16)] = jnp.ones((16,), jnp.float32)
        plsc.subcore_barrier()

        tbl = table3.at[c]
        row0 = s * ROWS_PER_S

        def outer(g, carry):
            r = row0 + g * INNER
            pltpu.sync_copy(src2.at[pl.ds(r, INNER)], src_v)
            pltpu.sync_copy(dst2.at[pl.ds(r, INNER)], dst_v)
            cps = []
            for j in range(INNER):
                cps.append(pltpu.async_copy(
                    tbl.at[src_v.at[j]],
                    rows_v.at[pl.ds(j * CHUNK, CHUNK)], sem))
            for j in range(INNER):
                cps[j].wait()
                pltpu.sync_copy(rows_v.at[pl.ds(j * CHUNK, CHUNK)],
                                agg_sh.at[dst_v.at[j]], add=True)
                if with_deg:
                    @pl.when(c == 0)
                    def _():
                        pltpu.sync_copy(ones_v, deg_sh.at[dst_v.at[j]],
                                        add=True)
            return carry

        lax.fori_loop(0, OUTER, outer, 0)
        plsc.subcore_barrier()
        pltpu.sync_copy(agg_sh.at[pl.ds(s * RPS, RPS)],
                        out_agg.at[c, pl.ds(s * RPS, RPS)])
        if with_deg:
            @pl.when(c == 0)
            def _():
                pltpu.sync_copy(deg_sh.at[pl.ds(s * RPS, RPS)],
                                out_deg.at[pl.ds(s * RPS, RPS)])

    return pl.kernel(body, out_type=tuple(out_type) if with_deg else out_type[0],
                     mesh=mesh, scratch_types=scratch,
                     compiler_params=pltpu.CompilerParams(
                         use_tc_tiling_on_sc=False))


_SC_AGG_DEG = _make_sc_agg(True)
_SC_AGG = _make_sc_agg(False)


def _make_combine(relu, split_out):
    R = 1024
    G = NP // R

    def body(x_ref, a_ref, d_ref, ws_ref, wn_ref, b_ref, o_ref):
        xcat = jnp.concatenate([x_ref[0], x_ref[1]], axis=1)
        deg = jnp.maximum(d_ref[...], 1.0)
        mean = jnp.concatenate([a_ref[0], a_ref[1]], axis=1) / deg
        y = (jnp.dot(xcat, ws_ref[...], preferred_element_type=jnp.float32)
             + jnp.dot(mean, wn_ref[...], preferred_element_type=jnp.float32)
             + b_ref[...])
        if relu:
            y = jnp.maximum(y, 0.0)
        if split_out:
            o_ref[0] = y[:, :SPL]
            o_ref[1] = y[:, SPL:]
        else:
            o_ref[...] = y

    if split_out:
        out_shape = jax.ShapeDtypeStruct((NC, NP, SPL), jnp.float32)
        out_spec = pl.BlockSpec((NC, R, SPL), lambda i: (0, i, 0))
    else:
        out_shape = jax.ShapeDtypeStruct((NP, D), jnp.float32)
        out_spec = pl.BlockSpec((R, D), lambda i: (i, 0))

    return pl.pallas_call(
        body,
        grid=(G,),
        in_specs=[
            pl.BlockSpec((NC, R, SPL), lambda i: (0, i, 0)),
            pl.BlockSpec((NC, R, SPL), lambda i: (0, i, 0)),
            pl.BlockSpec((R, 1), lambda i: (i, 0)),
            pl.BlockSpec((D, D), lambda i: (0, 0)),
            pl.BlockSpec((D, D), lambda i: (0, 0)),
            pl.BlockSpec((1, D), lambda i: (0, 0)),
        ],
        out_specs=out_spec,
        out_shape=out_shape,
    )


_COMBINE_RELU_SPLIT = _make_combine(True, True)
_COMBINE_PLAIN = _make_combine(False, False)


def kernel(x, edge_index, W1_self, W1_neigh, b1, W2_self, W2_neigh, b2):
    x = x.astype(jnp.float32)
    ei = edge_index.astype(jnp.int32)
    src2 = jnp.concatenate(
        [ei[0], jnp.zeros((EP - E,), jnp.int32)]).reshape(EP // CHUNK, CHUNK)
    dst2 = jnp.concatenate(
        [ei[1], jnp.full((EP - E,), NP - 1, jnp.int32)]).reshape(EP // CHUNK, CHUNK)
    xp = jnp.pad(x, ((0, NP - N), (0, 0)))
    xp3 = jnp.stack([xp[:, :SPL], xp[:, SPL:]])
    zrows = jnp.zeros((RPS, SPL), jnp.float32)
    zdeg = jnp.zeros((RPS,), jnp.float32)

    agg1, deg = _SC_AGG_DEG(xp3, src2, dst2, zrows, zdeg)
    dcol = deg[:, None]
    h3 = _COMBINE_RELU_SPLIT(xp3, agg1, dcol, W1_self, W1_neigh,
                             b1.reshape(1, D))
    agg2 = _SC_AGG(h3, src2, dst2, zrows)
    out = _COMBINE_PLAIN(h3, agg2, dcol, W2_self, W2_neigh,
                         b2.reshape(1, D))
    return out[:N]


# speedup vs baseline: 5.2740x; 1.0749x over previous
import jax
import jax.numpy as jnp
from jax import lax
from jax.experimental import pallas as pl
from jax.experimental.pallas import tpu as pltpu
from jax.experimental.pallas import tpu_sc as plsc

N = 10000
E = 320000
D = 128
SPL = 64
NP = 10240
NC = 2
NS = 16
CHUNK = 128
INNER = 5
EP = 327680
ROWS_PER_S = EP // NS // CHUNK
OUTER = ROWS_PER_S // INNER
RPS = NP // NS


def _make_sc_agg(with_deg):
    out_type = [jax.ShapeDtypeStruct((NC, NP, SPL), jnp.float32)]
    scratch = [
        pltpu.VMEM((ROWS_PER_S, CHUNK), jnp.int32),
        pltpu.VMEM((ROWS_PER_S, CHUNK), jnp.int32),
        pltpu.VMEM((INNER * CHUNK, SPL), jnp.float32),
        pltpu.VMEM_SHARED((NP, SPL), jnp.float32),
        pltpu.SemaphoreType.DMA,
        pltpu.SemaphoreType.DMA,
    ]
    if with_deg:
        out_type.append(jax.ShapeDtypeStruct((NP,), jnp.float32))
        scratch += [
            pltpu.VMEM((CHUNK,), jnp.float32),
            pltpu.VMEM_SHARED((NP,), jnp.float32),
            pltpu.SemaphoreType.DMA,
        ]

    mesh = plsc.VectorSubcoreMesh(core_axis_name="c", subcore_axis_name="s")

    def body(*refs):
        if with_deg:
            (table3, src2, dst2, zrows, zdeg, out_agg, out_deg,
             src_all, dst_all, rows_v, agg_sh, sem_g, sem_s,
             ones_v, deg_sh, sem_d) = refs
        else:
            (table3, src2, dst2, zrows, out_agg,
             src_all, dst_all, rows_v, agg_sh, sem_g, sem_s) = refs
        c = lax.axis_index("c")
        s = lax.axis_index("s")

        pltpu.sync_copy(zrows, agg_sh.at[pl.ds(s * RPS, RPS)])
        pltpu.sync_copy(src2.at[pl.ds(s * ROWS_PER_S, ROWS_PER_S)], src_all)
        pltpu.sync_copy(dst2.at[pl.ds(s * ROWS_PER_S, ROWS_PER_S)], dst_all)
        if with_deg:
            pltpu.sync_copy(zdeg, deg_sh.at[pl.ds(s * RPS, RPS)])
            for i in range(CHUNK // 16):
                ones_v[pl.ds(i * 16, 16)] = jnp.ones((16,), jnp.float32)
        plsc.subcore_barrier()

        tbl = table3.at[c]

        def outer(g, carry):
            k0 = g * INNER
            cps = []
            for j in range(INNER):
                cps.append(pltpu.async_copy(
                    tbl.at[src_all.at[k0 + j]],
                    rows_v.at[pl.ds(j * CHUNK, CHUNK)], sem_g))
            scs = []
            for j in range(INNER):
                cps[j].wait()
                scs.append(pltpu.async_copy(
                    rows_v.at[pl.ds(j * CHUNK, CHUNK)],
                    agg_sh.at[dst_all.at[k0 + j]], sem_s, add=True))
                if with_deg:
                    @pl.when(c == 0)
                    def _():
                        pltpu.async_copy(ones_v, deg_sh.at[dst_all.at[k0 + j]],
                                         sem_d, add=True)
            for sc in scs:
                sc.wait()
            if with_deg:
                @pl.when(c == 0)
                def _():
                    for j in range(INNER):
                        pltpu.make_async_copy(
                            ones_v, deg_sh.at[dst_all.at[k0 + j]],
                            sem_d).wait()
            return carry

        lax.fori_loop(0, OUTER, outer, 0)
        plsc.subcore_barrier()
        pltpu.sync_copy(agg_sh.at[pl.ds(s * RPS, RPS)],
                        out_agg.at[c, pl.ds(s * RPS, RPS)])
        if with_deg:
            @pl.when(c == 0)
            def _():
                pltpu.sync_copy(deg_sh.at[pl.ds(s * RPS, RPS)],
                                out_deg.at[pl.ds(s * RPS, RPS)])

    return pl.kernel(body, out_type=tuple(out_type) if with_deg else out_type[0],
                     mesh=mesh, scratch_types=scratch,
                     compiler_params=pltpu.CompilerParams(
                         use_tc_tiling_on_sc=False))


_SC_AGG_DEG = _make_sc_agg(True)
_SC_AGG = _make_sc_agg(False)


def _make_combine(relu, split_out):
    R = 1024
    G = NP // R

    def body(x_ref, a_ref, d_ref, ws_ref, wn_ref, b_ref, o_ref):
        xcat = jnp.concatenate([x_ref[0], x_ref[1]], axis=1)
        deg = jnp.maximum(d_ref[...], 1.0)
        mean = jnp.concatenate([a_ref[0], a_ref[1]], axis=1) / deg
        y = (jnp.dot(xcat, ws_ref[...], preferred_element_type=jnp.float32)
             + jnp.dot(mean, wn_ref[...], preferred_element_type=jnp.float32)
             + b_ref[...])
        if relu:
            y = jnp.maximum(y, 0.0)
        if split_out:
            o_ref[0] = y[:, :SPL]
            o_ref[1] = y[:, SPL:]
        else:
            o_ref[...] = y

    if split_out:
        out_shape = jax.ShapeDtypeStruct((NC, NP, SPL), jnp.float32)
        out_spec = pl.BlockSpec((NC, R, SPL), lambda i: (0, i, 0))
    else:
        out_shape = jax.ShapeDtypeStruct((NP, D), jnp.float32)
        out_spec = pl.BlockSpec((R, D), lambda i: (i, 0))

    return pl.pallas_call(
        body,
        grid=(G,),
        in_specs=[
            pl.BlockSpec((NC, R, SPL), lambda i: (0, i, 0)),
            pl.BlockSpec((NC, R, SPL), lambda i: (0, i, 0)),
            pl.BlockSpec((R, 1), lambda i: (i, 0)),
            pl.BlockSpec((D, D), lambda i: (0, 0)),
            pl.BlockSpec((D, D), lambda i: (0, 0)),
            pl.BlockSpec((1, D), lambda i: (0, 0)),
        ],
        out_specs=out_spec,
        out_shape=out_shape,
    )


_COMBINE_RELU_SPLIT = _make_combine(True, True)
_COMBINE_PLAIN = _make_combine(False, False)


def kernel(x, edge_index, W1_self, W1_neigh, b1, W2_self, W2_neigh, b2):
    x = x.astype(jnp.float32)
    ei = edge_index.astype(jnp.int32)
    src2 = jnp.concatenate(
        [ei[0], jnp.zeros((EP - E,), jnp.int32)]).reshape(EP // CHUNK, CHUNK)
    dst2 = jnp.concatenate(
        [ei[1], jnp.full((EP - E,), NP - 1, jnp.int32)]).reshape(EP // CHUNK, CHUNK)
    xp = jnp.pad(x, ((0, NP - N), (0, 0)))
    xp3 = jnp.stack([xp[:, :SPL], xp[:, SPL:]])
    zrows = jnp.zeros((RPS, SPL), jnp.float32)
    zdeg = jnp.zeros((RPS,), jnp.float32)

    agg1, deg = _SC_AGG_DEG(xp3, src2, dst2, zrows, zdeg)
    dcol = deg[:, None]
    h3 = _COMBINE_RELU_SPLIT(xp3, agg1, dcol, W1_self, W1_neigh,
                             b1.reshape(1, D))
    agg2 = _SC_AGG(h3, src2, dst2, zrows)
    out = _COMBINE_PLAIN(h3, agg2, dcol, W2_self, W2_neigh,
                         b2.reshape(1, D))
    return out[:N]


# speedup vs baseline: 5.5157x; 1.0458x over previous
import jax
import jax.numpy as jnp
from jax import lax
from jax.experimental import pallas as pl
from jax.experimental.pallas import tpu as pltpu
from jax.experimental.pallas import tpu_sc as plsc

N = 10000
E = 320000
D = 128
SPL = 64
NP = 10240
NC = 2
NS = 16
CHUNK = 128
INNER = 4
EP = 327680
ROWS_PER_S = EP // NS // CHUNK
HROWS = ROWS_PER_S // 2
BLOCKS_H = HROWS // INNER
TSTEPS = BLOCKS_H // 2
RPS = NP // NS


def _make_sc_agg(with_deg):
    out_type = [jax.ShapeDtypeStruct((NC, NP, SPL), jnp.float32)]
    scratch = [
        pltpu.VMEM((HROWS, CHUNK), jnp.int32),
        pltpu.VMEM((HROWS, CHUNK), jnp.int32),
        pltpu.VMEM((INNER * CHUNK, SPL), jnp.float32),
        pltpu.VMEM((INNER * CHUNK, SPL), jnp.float32),
        pltpu.VMEM_SHARED((NP, SPL), jnp.float32),
        pltpu.SemaphoreType.DMA,
        pltpu.SemaphoreType.DMA,
    ]
    if with_deg:
        out_type.append(jax.ShapeDtypeStruct((NC, NP), jnp.float32))
        scratch += [
            pltpu.VMEM((CHUNK,), jnp.float32),
            pltpu.VMEM_SHARED((NP,), jnp.float32),
            pltpu.SemaphoreType.DMA,
        ]

    mesh = plsc.VectorSubcoreMesh(core_axis_name="c", subcore_axis_name="s")

    def body(*refs):
        if with_deg:
            (table3, src2, dst2, zrows, zdeg, out_agg, out_deg,
             src_h, dst_h, rows0, rows1, agg_sh, sem_g, sem_s,
             ones_v, deg_sh, sem_d) = refs
        else:
            (table3, src2, dst2, zrows, out_agg,
             src_h, dst_h, rows0, rows1, agg_sh, sem_g, sem_s) = refs
        c = lax.axis_index("c")
        s = lax.axis_index("s")

        pltpu.sync_copy(zrows, agg_sh.at[pl.ds(s * RPS, RPS)])
        if with_deg:
            pltpu.sync_copy(zdeg, deg_sh.at[pl.ds(s * RPS, RPS)])
            for i in range(CHUNK // 16):
                ones_v[pl.ds(i * 16, 16)] = jnp.ones((16,), jnp.float32)
        plsc.subcore_barrier()

        tbl = table3.at[c]

        def fire_gathers(rows_buf, k0):
            for j in range(INNER):
                pltpu.async_copy(tbl.at[src_h.at[k0 + j]],
                                 rows_buf.at[pl.ds(j * CHUNK, CHUNK)], sem_g)

        def wait_gathers(rows_buf, k0):
            for j in range(INNER):
                pltpu.make_async_copy(
                    tbl.at[src_h.at[k0 + j]],
                    rows_buf.at[pl.ds(j * CHUNK, CHUNK)], sem_g).wait()

        def fire_scatters(rows_buf, k0, deg_core):
            for j in range(INNER):
                pltpu.async_copy(rows_buf.at[pl.ds(j * CHUNK, CHUNK)],
                                 agg_sh.at[dst_h.at[k0 + j]], sem_s, add=True)
            if with_deg:
                @pl.when(c == deg_core)
                def _():
                    for j in range(INNER):
                        pltpu.async_copy(ones_v, deg_sh.at[dst_h.at[k0 + j]],
                                         sem_d, add=True)

        def wait_scatters(rows_buf, k0, deg_core):
            for j in range(INNER):
                pltpu.make_async_copy(
                    rows_buf.at[pl.ds(j * CHUNK, CHUNK)],
                    agg_sh.at[dst_h.at[k0 + j]], sem_s).wait()
            if with_deg:
                @pl.when(c == deg_core)
                def _():
                    for j in range(INNER):
                        pltpu.make_async_copy(
                            ones_v, deg_sh.at[dst_h.at[k0 + j]], sem_d).wait()

        for half in range(2):
            r0 = s * ROWS_PER_S + half * HROWS
            pltpu.sync_copy(src2.at[pl.ds(r0, HROWS)], src_h)
            pltpu.sync_copy(dst2.at[pl.ds(r0, HROWS)], dst_h)
            fire_gathers(rows0, 0)

            def tbody(t, carry):
                ka = 2 * t * INNER
                kb = ka + INNER

                @pl.when(t > 0)
                def _():
                    wait_scatters(rows1, ka - INNER, 1)
                fire_gathers(rows1, kb)
                wait_gathers(rows0, ka)
                fire_scatters(rows0, ka, 0)
                wait_scatters(rows0, ka, 0)

                @pl.when(t < TSTEPS - 1)
                def _():
                    fire_gathers(rows0, kb + INNER)
                wait_gathers(rows1, kb)
                fire_scatters(rows1, kb, 1)
                return carry

            lax.fori_loop(0, TSTEPS, tbody, 0)
            wait_scatters(rows1, (BLOCKS_H - 1) * INNER, 1)

        plsc.subcore_barrier()
        pltpu.sync_copy(agg_sh.at[pl.ds(s * RPS, RPS)],
                        out_agg.at[c, pl.ds(s * RPS, RPS)])
        if with_deg:
            pltpu.sync_copy(deg_sh.at[pl.ds(s * RPS, RPS)],
                            out_deg.at[c, pl.ds(s * RPS, RPS)])

    return pl.kernel(body, out_type=tuple(out_type) if with_deg else out_type[0],
                     mesh=mesh, scratch_types=scratch,
                     compiler_params=pltpu.CompilerParams(
                         use_tc_tiling_on_sc=False))


_SC_AGG_DEG = _make_sc_agg(True)
_SC_AGG = _make_sc_agg(False)


def _make_combine(relu, split_out):
    R = 1024
    G = NP // R

    def body(x_ref, a_ref, d0_ref, d1_ref, ws_ref, wn_ref, b_ref, o_ref):
        xcat = jnp.concatenate([x_ref[0], x_ref[1]], axis=1)
        deg = jnp.maximum(d0_ref[...] + d1_ref[...], 1.0)
        mean = jnp.concatenate([a_ref[0], a_ref[1]], axis=1) / deg
        y = (jnp.dot(xcat, ws_ref[...], preferred_element_type=jnp.float32)
             + jnp.dot(mean, wn_ref[...], preferred_element_type=jnp.float32)
             + b_ref[...])
        if relu:
            y = jnp.maximum(y, 0.0)
        if split_out:
            o_ref[0] = y[:, :SPL]
            o_ref[1] = y[:, SPL:]
        else:
            o_ref[...] = y

    if split_out:
        out_shape = jax.ShapeDtypeStruct((NC, NP, SPL), jnp.float32)
        out_spec = pl.BlockSpec((NC, R, SPL), lambda i: (0, i, 0))
    else:
        out_shape = jax.ShapeDtypeStruct((NP, D), jnp.float32)
        out_spec = pl.BlockSpec((R, D), lambda i: (i, 0))

    return pl.pallas_call(
        body,
        grid=(G,),
        in_specs=[
            pl.BlockSpec((NC, R, SPL), lambda i: (0, i, 0)),
            pl.BlockSpec((NC, R, SPL), lambda i: (0, i, 0)),
            pl.BlockSpec((R, 1), lambda i: (i, 0)),
            pl.BlockSpec((R, 1), lambda i: (i, 0)),
            pl.BlockSpec((D, D), lambda i: (0, 0)),
            pl.BlockSpec((D, D), lambda i: (0, 0)),
            pl.BlockSpec((1, D), lambda i: (0, 0)),
        ],
        out_specs=out_spec,
        out_shape=out_shape,
    )


_COMBINE_RELU_SPLIT = _make_combine(True, True)
_COMBINE_PLAIN = _make_combine(False, False)


def kernel(x, edge_index, W1_self, W1_neigh, b1, W2_self, W2_neigh, b2):
    x = x.astype(jnp.float32)
    ei = edge_index.astype(jnp.int32)
    src2 = jnp.concatenate(
        [ei[0], jnp.zeros((EP - E,), jnp.int32)]).reshape(EP // CHUNK, CHUNK)
    dst2 = jnp.concatenate(
        [ei[1], jnp.full((EP - E,), NP - 1, jnp.int32)]).reshape(EP // CHUNK, CHUNK)
    xp = jnp.pad(x, ((0, NP - N), (0, 0)))
    xp3 = jnp.stack([xp[:, :SPL], xp[:, SPL:]])
    zrows = jnp.zeros((RPS, SPL), jnp.float32)
    zdeg = jnp.zeros((RPS,), jnp.float32)

    agg1, deg = _SC_AGG_DEG(xp3, src2, dst2, zrows, zdeg)
    d0 = deg[0][:, None]
    d1 = deg[1][:, None]
    h3 = _COMBINE_RELU_SPLIT(xp3, agg1, d0, d1, W1_self, W1_neigh,
                             b1.reshape(1, D))
    agg2 = _SC_AGG(h3, src2, dst2, zrows)
    out = _COMBINE_PLAIN(h3, agg2, d0, d1, W2_self, W2_neigh,
                         b2.reshape(1, D))
    return out[:N]
